# Initial kernel scaffold; baseline (speedup 1.0000x reference)
import functools

import jax
import jax.numpy as jnp
from jax import lax
from jax.experimental import pallas as pl
from jax.experimental.pallas import tpu as pltpu
from jax.experimental.pallas import tpu_sc as plsc

_NTILE = 32
_K = 80
_ZR = 125


def _scatter_rows(n, e, w):
    ntile = _NTILE
    ept = e // ntile
    nch = ept // _K
    rpt = n // 16
    nz = rpt // _ZR
    mesh = plsc.VectorSubcoreMesh(core_axis_name="c", subcore_axis_name="s")

    @functools.partial(
        pl.kernel,
        mesh=mesh,
        out_type=jax.ShapeDtypeStruct((2, n, w), jnp.float32),
        scratch_types=[
            pltpu.VMEM((nch, _K), jnp.int32),
            pltpu.VMEM((nch, _K), jnp.int32),
            pltpu.VMEM((_K, w), jnp.float32),
            pltpu.VMEM((_ZR, w), jnp.float32),
            pltpu.VMEM_SHARED((n, w), jnp.float32),
            pltpu.SemaphoreType.DMA,
        ],
    )
    def k(g_hbm, src_hbm, dst_hbm, out_hbm, src_v, dst_v, rows_v, zer_v, acc_sh, gsem):
        c = lax.axis_index("c")
        s = lax.axis_index("s")
        wid = c * 16 + s

        def zero_row(i, carry):
            for j in range(w // 16):
                zer_v[i, pl.ds(j * 16, 16)] = jnp.zeros((16,), jnp.float32)
            return carry
        lax.fori_loop(0, _ZR, zero_row, 0)

        base = s * rpt
        for r in range(nz):
            pltpu.sync_copy(zer_v, acc_sh.at[pl.ds(base + r * _ZR, _ZR)])
        plsc.subcore_barrier()

        pltpu.sync_copy(src_hbm.at[wid], src_v)
        pltpu.sync_copy(dst_hbm.at[wid], dst_v)

        def edge_chunk(i, carry):
            pltpu.async_copy(g_hbm.at[src_v.at[i]], rows_v, gsem).wait()
            pltpu.sync_copy(rows_v, acc_sh.at[dst_v.at[i]], add=True)
            return carry
        lax.fori_loop(0, nch, edge_chunk, 0)

        plsc.subcore_barrier()
        for r in range(nz):
            pltpu.sync_copy(acc_sh.at[pl.ds(base + r * _ZR, _ZR)],
                            out_hbm.at[c, pl.ds(base + r * _ZR, _ZR)])

    return k


def _deg_rows(n, e, w):
    ntile = _NTILE
    ept = e // ntile
    nch = ept // _K
    rpt = n // 16
    nz = rpt // _ZR
    mesh = plsc.VectorSubcoreMesh(core_axis_name="c", subcore_axis_name="s")

    @functools.partial(
        pl.kernel,
        mesh=mesh,
        out_type=jax.ShapeDtypeStruct((2, n, w), jnp.float32),
        scratch_types=[
            pltpu.VMEM((nch, _K), jnp.int32),
            pltpu.VMEM((_K, w), jnp.float32),
            pltpu.VMEM((_ZR, w), jnp.float32),
            pltpu.VMEM_SHARED((n, w), jnp.float32),
        ],
    )
    def k(dst_hbm, out_hbm, dst_v, ones_v, zer_v, acc_sh):
        c = lax.axis_index("c")
        s = lax.axis_index("s")
        wid = c * 16 + s

        def zero_row(i, carry):
            for j in range(w // 16):
                zer_v[i, pl.ds(j * 16, 16)] = jnp.zeros((16,), jnp.float32)
            return carry
        lax.fori_loop(0, _ZR, zero_row, 0)

        def one_row(i, carry):
            for j in range(w // 16):
                ones_v[i, pl.ds(j * 16, 16)] = jnp.ones((16,), jnp.float32)
            return carry
        lax.fori_loop(0, _K, one_row, 0)

        base = s * rpt
        for r in range(nz):
            pltpu.sync_copy(zer_v, acc_sh.at[pl.ds(base + r * _ZR, _ZR)])
        plsc.subcore_barrier()

        pltpu.sync_copy(dst_hbm.at[wid], dst_v)

        def edge_chunk(i, carry):
            pltpu.sync_copy(ones_v, acc_sh.at[dst_v.at[i]], add=True)
            return carry
        lax.fori_loop(0, nch, edge_chunk, 0)

        plsc.subcore_barrier()
        for r in range(nz):
            pltpu.sync_copy(acc_sh.at[pl.ds(base + r * _ZR, _ZR)],
                            out_hbm.at[c, pl.ds(base + r * _ZR, _ZR)])

    return k


def _tc_prep(degp, x, W1):
    n = x.shape[0]
    h = W1.shape[1]

    def body(degp_ref, x_ref, w1_ref, dinv_ref, g1_ref):
        deg = degp_ref[0, :, 0:1] + degp_ref[1, :, 0:1] + 1.0
        dinv = lax.rsqrt(deg)
        dinv_ref[...] = dinv
        g1_ref[...] = dinv * jnp.dot(x_ref[...], w1_ref[...],
                                     preferred_element_type=jnp.float32)

    return pl.pallas_call(
        body,
        out_shape=(jax.ShapeDtypeStruct((n, 1), jnp.float32),
                   jax.ShapeDtypeStruct((n, h), jnp.float32)),
    )(degp, x, W1)


def _tc_mid(sp, g, dinv, b, gam, bet, W):
    n, h = g.shape
    hn = W.shape[1]

    def body(sp_ref, g_ref, dinv_ref, b_ref, gam_ref, bet_ref, w_ref, gout_ref):
        dinv = dinv_ref[...]
        t = dinv * (sp_ref[0] + sp_ref[1] + g_ref[...]) + b_ref[...]
        mu = jnp.mean(t, axis=0, keepdims=True)
        xc = t - mu
        var = jnp.mean(xc * xc, axis=0, keepdims=True)
        hh = jnp.maximum(gam_ref[...] * xc * lax.rsqrt(var + 1e-5) + bet_ref[...], 0.0)
        gout_ref[...] = dinv * jnp.dot(hh, w_ref[...],
                                       preferred_element_type=jnp.float32)

    return pl.pallas_call(
        body,
        out_shape=jax.ShapeDtypeStruct((n, hn), jnp.float32),
    )(sp, g, dinv, b, gam, bet, W)


def _tc_final(sp, g, dinv, b, gam, bet, batch2, Wl1, bl1, Wl2, bl2, G):
    n, h = g.shape
    out = Wl2.shape[1]

    def body(sp_ref, g_ref, dinv_ref, b_ref, gam_ref, bet_ref, batch_ref,
             wl1_ref, bl1_ref, wl2_ref, bl2_ref, out_ref):
        dinv = dinv_ref[...]
        t = dinv * (sp_ref[0] + sp_ref[1] + g_ref[...]) + b_ref[...]
        mu = jnp.mean(t, axis=0, keepdims=True)
        xc = t - mu
        var = jnp.mean(xc * xc, axis=0, keepdims=True)
        hh = jnp.maximum(gam_ref[...] * xc * lax.rsqrt(var + 1e-5) + bet_ref[...], 0.0)

        gids = lax.broadcasted_iota(jnp.int32, (1, G), 1)
        m = (batch_ref[...] == gids).astype(jnp.float32)
        psum = lax.dot_general(m, hh, (((0,), (0,)), ((), ())),
                               preferred_element_type=jnp.float32)
        cnt = lax.dot_general(m, jnp.ones((n, 1), jnp.float32),
                              (((0,), (0,)), ((), ())),
                              preferred_element_type=jnp.float32)
        pooled = psum / jnp.maximum(cnt, 1.0)
        z = jnp.maximum(jnp.dot(pooled, wl1_ref[...],
                                preferred_element_type=jnp.float32) + bl1_ref[...], 0.0)
        out_ref[...] = jnp.dot(z, wl2_ref[...],
                               preferred_element_type=jnp.float32) + bl2_ref[...]

    return pl.pallas_call(
        body,
        out_shape=jax.ShapeDtypeStruct((G, out), jnp.float32),
    )(sp, g, dinv, b, gam, bet, batch2, Wl1, bl1, Wl2, bl2)


def kernel(x, edge_index, batch, W1, b1, g1, beta1, W2, b2, g2, beta2,
           W3, b3, g3, beta3, Wl1, bl1, Wl2, bl2):
    n, din = x.shape
    e = edge_index.shape[1]
    h = W1.shape[1]
    G = 64

    ept = e // _NTILE
    nch = ept // _K
    src3 = edge_index[0].reshape(_NTILE, nch, _K)
    dst3 = edge_index[1].reshape(_NTILE, nch, _K)

    degp = _deg_rows(n, e, 16)(dst3)
    dinv, gcur = _tc_prep(degp, x, W1)

    scat = _scatter_rows(n, e, h)
    b1r, gam1, bet1 = b1.reshape(1, h), g1.reshape(1, h), beta1.reshape(1, h)
    b2r, gam2, bet2 = b2.reshape(1, h), g2.reshape(1, h), beta2.reshape(1, h)
    b3r, gam3, bet3 = b3.reshape(1, h), g3.reshape(1, h), beta3.reshape(1, h)

    sp = scat(gcur, src3, dst3)
    gcur = _tc_mid(sp, gcur, dinv, b1r, gam1, bet1, W2)
    sp = scat(gcur, src3, dst3)
    gcur = _tc_mid(sp, gcur, dinv, b2r, gam2, bet2, W3)
    sp = scat(gcur, src3, dst3)

    batch2 = batch.reshape(n, 1)
    return _tc_final(sp, gcur, dinv, b3r, gam3, bet3, batch2,
                     Wl1, bl1.reshape(1, -1), Wl2, bl2.reshape(1, -1), G)


# speedup vs baseline: 20.7829x; 20.7829x over previous
import functools

import jax
import jax.numpy as jnp
from jax import lax
from jax.experimental import pallas as pl
from jax.experimental.pallas import tpu as pltpu
from jax.experimental.pallas import tpu_sc as plsc

_NTILE = 32
_K = 80
_ZR = 128


def _pad_rows(n):
    return -(-n // (16 * _ZR)) * (16 * _ZR)


def _scatter_rows(n, e, w):
    ntile = _NTILE
    ept = e // ntile
    nch = ept // _K
    npad = _pad_rows(n)
    rpt = npad // 16
    nz = rpt // _ZR
    mesh = plsc.VectorSubcoreMesh(core_axis_name="c", subcore_axis_name="s")

    @functools.partial(
        pl.kernel,
        mesh=mesh,
        out_type=jax.ShapeDtypeStruct((2, npad, w), jnp.float32),
        scratch_types=[
            pltpu.VMEM((nch, _K), jnp.int32),
            pltpu.VMEM((nch, _K), jnp.int32),
            pltpu.VMEM((_K, w), jnp.float32),
            pltpu.VMEM((_ZR, w), jnp.float32),
            pltpu.VMEM_SHARED((npad, w), jnp.float32),
            pltpu.SemaphoreType.DMA,
        ],
        compiler_params=pltpu.CompilerParams(use_tc_tiling_on_sc=False),
    )
    def k(g_hbm, src_hbm, dst_hbm, out_hbm, src_v, dst_v, rows_v, zer_v, acc_sh, gsem):
        c = lax.axis_index("c")
        s = lax.axis_index("s")
        wid = c * 16 + s

        def zero_row(i, carry):
            for j in range(w // 16):
                zer_v[i, pl.ds(j * 16, 16)] = jnp.zeros((16,), jnp.float32)
            return carry
        lax.fori_loop(0, _ZR, zero_row, 0)

        base = s * rpt
        for r in range(nz):
            pltpu.sync_copy(zer_v, acc_sh.at[pl.ds(base + r * _ZR, _ZR)])
        plsc.subcore_barrier()

        pltpu.sync_copy(src_hbm.at[wid], src_v)
        pltpu.sync_copy(dst_hbm.at[wid], dst_v)

        def edge_chunk(i, carry):
            pltpu.async_copy(g_hbm.at[src_v.at[i]], rows_v, gsem).wait()
            pltpu.sync_copy(rows_v, acc_sh.at[dst_v.at[i]], add=True)
            return carry
        lax.fori_loop(0, nch, edge_chunk, 0)

        plsc.subcore_barrier()
        for r in range(nz):
            pltpu.sync_copy(acc_sh.at[pl.ds(base + r * _ZR, _ZR)],
                            out_hbm.at[c, pl.ds(base + r * _ZR, _ZR)])

    return k


def _deg_rows(n, e, w):
    ntile = _NTILE
    ept = e // ntile
    nch = ept // _K
    npad = _pad_rows(n)
    rpt = npad // 16
    nz = rpt // _ZR
    mesh = plsc.VectorSubcoreMesh(core_axis_name="c", subcore_axis_name="s")

    @functools.partial(
        pl.kernel,
        mesh=mesh,
        out_type=jax.ShapeDtypeStruct((2, npad, w), jnp.float32),
        scratch_types=[
            pltpu.VMEM((nch, _K), jnp.int32),
            pltpu.VMEM((_K, w), jnp.float32),
            pltpu.VMEM((_ZR, w), jnp.float32),
            pltpu.VMEM_SHARED((npad, w), jnp.float32),
        ],
        compiler_params=pltpu.CompilerParams(use_tc_tiling_on_sc=False),
    )
    def k(dst_hbm, out_hbm, dst_v, ones_v, zer_v, acc_sh):
        c = lax.axis_index("c")
        s = lax.axis_index("s")
        wid = c * 16 + s

        def zero_row(i, carry):
            for j in range(w // 16):
                zer_v[i, pl.ds(j * 16, 16)] = jnp.zeros((16,), jnp.float32)
            return carry
        lax.fori_loop(0, _ZR, zero_row, 0)

        def one_row(i, carry):
            for j in range(w // 16):
                ones_v[i, pl.ds(j * 16, 16)] = jnp.ones((16,), jnp.float32)
            return carry
        lax.fori_loop(0, _K, one_row, 0)

        base = s * rpt
        for r in range(nz):
            pltpu.sync_copy(zer_v, acc_sh.at[pl.ds(base + r * _ZR, _ZR)])
        plsc.subcore_barrier()

        pltpu.sync_copy(dst_hbm.at[wid], dst_v)

        def edge_chunk(i, carry):
            pltpu.sync_copy(ones_v, acc_sh.at[dst_v.at[i]], add=True)
            return carry
        lax.fori_loop(0, nch, edge_chunk, 0)

        plsc.subcore_barrier()
        for r in range(nz):
            pltpu.sync_copy(acc_sh.at[pl.ds(base + r * _ZR, _ZR)],
                            out_hbm.at[c, pl.ds(base + r * _ZR, _ZR)])

    return k


def _tc_prep(degp, x, W1):
    n = x.shape[0]
    h = W1.shape[1]

    def body(degp_ref, x_ref, w1_ref, dinv_ref, g1_ref):
        deg = degp_ref[0, 0:n, 0:1] + degp_ref[1, 0:n, 0:1] + 1.0
        dinv = lax.rsqrt(deg)
        dinv_ref[...] = dinv
        g1_ref[...] = dinv * jnp.dot(x_ref[...], w1_ref[...],
                                     preferred_element_type=jnp.float32)

    return pl.pallas_call(
        body,
        out_shape=(jax.ShapeDtypeStruct((n, 1), jnp.float32),
                   jax.ShapeDtypeStruct((n, h), jnp.float32)),
    )(degp, x, W1)


def _tc_mid(sp, g, dinv, b, gam, bet, W):
    n, h = g.shape
    hn = W.shape[1]

    def body(sp_ref, g_ref, dinv_ref, b_ref, gam_ref, bet_ref, w_ref, gout_ref):
        dinv = dinv_ref[...]
        t = dinv * (sp_ref[0, 0:n, :] + sp_ref[1, 0:n, :] + g_ref[...]) + b_ref[...]
        mu = jnp.mean(t, axis=0, keepdims=True)
        xc = t - mu
        var = jnp.mean(xc * xc, axis=0, keepdims=True)
        hh = jnp.maximum(gam_ref[...] * xc * lax.rsqrt(var + 1e-5) + bet_ref[...], 0.0)
        gout_ref[...] = dinv * jnp.dot(hh, w_ref[...],
                                       preferred_element_type=jnp.float32)

    return pl.pallas_call(
        body,
        out_shape=jax.ShapeDtypeStruct((n, hn), jnp.float32),
    )(sp, g, dinv, b, gam, bet, W)


def _tc_final(sp, g, dinv, b, gam, bet, batch2, Wl1, bl1, Wl2, bl2, G):
    n, h = g.shape
    out = Wl2.shape[1]

    def body(sp_ref, g_ref, dinv_ref, b_ref, gam_ref, bet_ref, batch_ref,
             wl1_ref, bl1_ref, wl2_ref, bl2_ref, out_ref):
        dinv = dinv_ref[...]
        t = dinv * (sp_ref[0, 0:n, :] + sp_ref[1, 0:n, :] + g_ref[...]) + b_ref[...]
        mu = jnp.mean(t, axis=0, keepdims=True)
        xc = t - mu
        var = jnp.mean(xc * xc, axis=0, keepdims=True)
        hh = jnp.maximum(gam_ref[...] * xc * lax.rsqrt(var + 1e-5) + bet_ref[...], 0.0)

        gids = lax.broadcasted_iota(jnp.int32, (1, G), 1)
        m = (batch_ref[...] == gids).astype(jnp.float32)
        psum = lax.dot_general(m, hh, (((0,), (0,)), ((), ())),
                               preferred_element_type=jnp.float32)
        cnt = lax.dot_general(m, jnp.ones((n, 1), jnp.float32),
                              (((0,), (0,)), ((), ())),
                              preferred_element_type=jnp.float32)
        pooled = psum / jnp.maximum(cnt, 1.0)
        z = jnp.maximum(jnp.dot(pooled, wl1_ref[...],
                                preferred_element_type=jnp.float32) + bl1_ref[...], 0.0)
        out_ref[...] = jnp.dot(z, wl2_ref[...],
                               preferred_element_type=jnp.float32) + bl2_ref[...]

    return pl.pallas_call(
        body,
        out_shape=jax.ShapeDtypeStruct((G, out), jnp.float32),
    )(sp, g, dinv, b, gam, bet, batch2, Wl1, bl1, Wl2, bl2)


def kernel(x, edge_index, batch, W1, b1, g1, beta1, W2, b2, g2, beta2,
           W3, b3, g3, beta3, Wl1, bl1, Wl2, bl2):
    n, din = x.shape
    e = edge_index.shape[1]
    h = W1.shape[1]
    G = 64

    ept = e // _NTILE
    nch = ept // _K
    src3 = edge_index[0].reshape(_NTILE, nch, _K)
    dst3 = edge_index[1].reshape(_NTILE, nch, _K)

    degp = _deg_rows(n, e, 16)(dst3)
    dinv, gcur = _tc_prep(degp, x, W1)

    scat = _scatter_rows(n, e, h)
    b1r, gam1, bet1 = b1.reshape(1, h), g1.reshape(1, h), beta1.reshape(1, h)
    b2r, gam2, bet2 = b2.reshape(1, h), g2.reshape(1, h), beta2.reshape(1, h)
    b3r, gam3, bet3 = b3.reshape(1, h), g3.reshape(1, h), beta3.reshape(1, h)

    sp = scat(gcur, src3, dst3)
    gcur = _tc_mid(sp, gcur, dinv, b1r, gam1, bet1, W2)
    sp = scat(gcur, src3, dst3)
    gcur = _tc_mid(sp, gcur, dinv, b2r, gam2, bet2, W3)
    sp = scat(gcur, src3, dst3)

    batch2 = batch.reshape(n, 1)
    return _tc_final(sp, gcur, dinv, b3r, gam3, bet3, batch2,
                     Wl1, bl1.reshape(1, -1), Wl2, bl2.reshape(1, -1), G)


# speedup vs baseline: 31.4527x; 1.5134x over previous
import functools

import jax
import jax.numpy as jnp
from jax import lax
from jax.experimental import pallas as pl
from jax.experimental.pallas import tpu as pltpu
from jax.experimental.pallas import tpu_sc as plsc

_NTILE = 32
_K = 80
_ZR = 128


def _pad_rows(n):
    return -(-n // (16 * _ZR)) * (16 * _ZR)


def _scatter_rows(n, e, w):
    ntile = _NTILE
    ept = e // ntile
    nch = ept // _K
    npad = _pad_rows(n)
    rpt = npad // 16
    nz = rpt // _ZR
    assert nch % 2 == 1, "edge loop expects an odd chunk count (paired + tail)"
    mesh = plsc.VectorSubcoreMesh(core_axis_name="c", subcore_axis_name="s")

    @functools.partial(
        pl.kernel,
        mesh=mesh,
        out_type=jax.ShapeDtypeStruct((2, npad, w), jnp.float32),
        scratch_types=[
            pltpu.VMEM((nch, _K), jnp.int32),
            pltpu.VMEM((nch, _K), jnp.int32),
            pltpu.VMEM((2, _K, w), jnp.float32),
            pltpu.VMEM((_ZR, w), jnp.float32),
            pltpu.VMEM_SHARED((npad, w), jnp.float32),
            pltpu.SemaphoreType.DMA,
            pltpu.SemaphoreType.DMA,
            pltpu.SemaphoreType.DMA,
        ],
        compiler_params=pltpu.CompilerParams(use_tc_tiling_on_sc=False),
    )
    def k(g_hbm, src_hbm, dst_hbm, out_hbm, src_v, dst_v, rows_v, zer_v, acc_sh,
          gsem0, gsem1, isem):
        c = lax.axis_index("c")
        s = lax.axis_index("s")
        wid = c * 16 + s

        icp0 = pltpu.async_copy(src_hbm.at[wid], src_v, isem)
        icp1 = pltpu.async_copy(dst_hbm.at[wid], dst_v, isem)

        def zero_row(i, carry):
            for j in range(w // 16):
                zer_v[i, pl.ds(j * 16, 16)] = jnp.zeros((16,), jnp.float32)
            return carry
        lax.fori_loop(0, _ZR, zero_row, 0)

        base = s * rpt
        for r in range(nz):
            pltpu.sync_copy(zer_v, acc_sh.at[pl.ds(base + r * _ZR, _ZR)])
        icp0.wait()
        icp1.wait()
        plsc.subcore_barrier()

        pltpu.async_copy(g_hbm.at[src_v.at[0]], rows_v.at[0], gsem0)

        def pair(p, carry):
            i0 = 2 * p
            pltpu.async_copy(g_hbm.at[src_v.at[i0 + 1]], rows_v.at[1], gsem1)
            pltpu.make_async_copy(g_hbm.at[pl.ds(0, _K)], rows_v.at[0], gsem0).wait()
            pltpu.sync_copy(rows_v.at[0], acc_sh.at[dst_v.at[i0]], add=True)
            pltpu.async_copy(g_hbm.at[src_v.at[i0 + 2]], rows_v.at[0], gsem0)
            pltpu.make_async_copy(g_hbm.at[pl.ds(0, _K)], rows_v.at[1], gsem1).wait()
            pltpu.sync_copy(rows_v.at[1], acc_sh.at[dst_v.at[i0 + 1]], add=True)
            return carry
        lax.fori_loop(0, nch // 2, pair, 0)

        pltpu.make_async_copy(g_hbm.at[pl.ds(0, _K)], rows_v.at[0], gsem0).wait()
        pltpu.sync_copy(rows_v.at[0], acc_sh.at[dst_v.at[nch - 1]], add=True)

        plsc.subcore_barrier()
        for r in range(nz):
            pltpu.sync_copy(acc_sh.at[pl.ds(base + r * _ZR, _ZR)],
                            out_hbm.at[c, pl.ds(base + r * _ZR, _ZR)])

    return k


def _deg_rows(n, e, w):
    ntile = _NTILE
    ept = e // ntile
    nch = ept // _K
    npad = _pad_rows(n)
    rpt = npad // 16
    nz = rpt // _ZR
    mesh = plsc.VectorSubcoreMesh(core_axis_name="c", subcore_axis_name="s")

    @functools.partial(
        pl.kernel,
        mesh=mesh,
        out_type=jax.ShapeDtypeStruct((2, npad, w), jnp.float32),
        scratch_types=[
            pltpu.VMEM((nch, _K), jnp.int32),
            pltpu.VMEM((_K, w), jnp.float32),
            pltpu.VMEM((_ZR, w), jnp.float32),
            pltpu.VMEM_SHARED((npad, w), jnp.float32),
            pltpu.SemaphoreType.DMA,
        ],
        compiler_params=pltpu.CompilerParams(use_tc_tiling_on_sc=False),
    )
    def k(dst_hbm, out_hbm, dst_v, ones_v, zer_v, acc_sh, isem):
        c = lax.axis_index("c")
        s = lax.axis_index("s")
        wid = c * 16 + s
        icp = pltpu.async_copy(dst_hbm.at[wid], dst_v, isem)

        def zero_row(i, carry):
            for j in range(w // 16):
                zer_v[i, pl.ds(j * 16, 16)] = jnp.zeros((16,), jnp.float32)
            return carry
        lax.fori_loop(0, _ZR, zero_row, 0)

        def one_row(i, carry):
            for j in range(w // 16):
                ones_v[i, pl.ds(j * 16, 16)] = jnp.ones((16,), jnp.float32)
            return carry
        lax.fori_loop(0, _K, one_row, 0)

        base = s * rpt
        for r in range(nz):
            pltpu.sync_copy(zer_v, acc_sh.at[pl.ds(base + r * _ZR, _ZR)])
        icp.wait()
        plsc.subcore_barrier()

        def edge_chunk(i, carry):
            pltpu.sync_copy(ones_v, acc_sh.at[dst_v.at[i]], add=True)
            return carry
        lax.fori_loop(0, nch, edge_chunk, 0)

        plsc.subcore_barrier()
        for r in range(nz):
            pltpu.sync_copy(acc_sh.at[pl.ds(base + r * _ZR, _ZR)],
                            out_hbm.at[c, pl.ds(base + r * _ZR, _ZR)])

    return k


def _tc_prep(degp, x, W1):
    n = x.shape[0]
    h = W1.shape[1]

    def body(degp_ref, x_ref, w1_ref, dinv_ref, g1_ref):
        deg = degp_ref[0, 0:n, 0:1] + degp_ref[1, 0:n, 0:1] + 1.0
        dinv = lax.rsqrt(deg)
        dinv_ref[...] = dinv
        g1_ref[...] = dinv * jnp.dot(x_ref[...], w1_ref[...],
                                     preferred_element_type=jnp.float32)

    return pl.pallas_call(
        body,
        out_shape=(jax.ShapeDtypeStruct((n, 1), jnp.float32),
                   jax.ShapeDtypeStruct((n, h), jnp.float32)),
    )(degp, x, W1)


def _tc_mid(sp, g, dinv, b, gam, bet, W):
    n, h = g.shape
    hn = W.shape[1]

    def body(sp_ref, g_ref, dinv_ref, b_ref, gam_ref, bet_ref, w_ref, gout_ref):
        dinv = dinv_ref[...]
        t = dinv * (sp_ref[0, 0:n, :] + sp_ref[1, 0:n, :] + g_ref[...]) + b_ref[...]
        mu = jnp.mean(t, axis=0, keepdims=True)
        xc = t - mu
        var = jnp.mean(xc * xc, axis=0, keepdims=True)
        hh = jnp.maximum(gam_ref[...] * xc * lax.rsqrt(var + 1e-5) + bet_ref[...], 0.0)
        gout_ref[...] = dinv * jnp.dot(hh, w_ref[...],
                                       preferred_element_type=jnp.float32)

    return pl.pallas_call(
        body,
        out_shape=jax.ShapeDtypeStruct((n, hn), jnp.float32),
    )(sp, g, dinv, b, gam, bet, W)


def _tc_final(sp, g, dinv, b, gam, bet, batch2, Wl1, bl1, Wl2, bl2, G):
    n, h = g.shape
    out = Wl2.shape[1]

    def body(sp_ref, g_ref, dinv_ref, b_ref, gam_ref, bet_ref, batch_ref,
             wl1_ref, bl1_ref, wl2_ref, bl2_ref, out_ref):
        dinv = dinv_ref[...]
        t = dinv * (sp_ref[0, 0:n, :] + sp_ref[1, 0:n, :] + g_ref[...]) + b_ref[...]
        mu = jnp.mean(t, axis=0, keepdims=True)
        xc = t - mu
        var = jnp.mean(xc * xc, axis=0, keepdims=True)
        hh = jnp.maximum(gam_ref[...] * xc * lax.rsqrt(var + 1e-5) + bet_ref[...], 0.0)

        gids = lax.broadcasted_iota(jnp.int32, (1, G), 1)
        m = (batch_ref[...] == gids).astype(jnp.float32)
        psum = lax.dot_general(m, hh, (((0,), (0,)), ((), ())),
                               preferred_element_type=jnp.float32)
        cnt = lax.dot_general(m, jnp.ones((n, 1), jnp.float32),
                              (((0,), (0,)), ((), ())),
                              preferred_element_type=jnp.float32)
        pooled = psum / jnp.maximum(cnt, 1.0)
        z = jnp.maximum(jnp.dot(pooled, wl1_ref[...],
                                preferred_element_type=jnp.float32) + bl1_ref[...], 0.0)
        out_ref[...] = jnp.dot(z, wl2_ref[...],
                               preferred_element_type=jnp.float32) + bl2_ref[...]

    return pl.pallas_call(
        body,
        out_shape=jax.ShapeDtypeStruct((G, out), jnp.float32),
    )(sp, g, dinv, b, gam, bet, batch2, Wl1, bl1, Wl2, bl2)


def kernel(x, edge_index, batch, W1, b1, g1, beta1, W2, b2, g2, beta2,
           W3, b3, g3, beta3, Wl1, bl1, Wl2, bl2):
    n, din = x.shape
    e = edge_index.shape[1]
    h = W1.shape[1]
    G = 64

    ept = e // _NTILE
    nch = ept // _K
    src3 = edge_index[0].reshape(_NTILE, nch, _K)
    dst3 = edge_index[1].reshape(_NTILE, nch, _K)

    degp = _deg_rows(n, e, 16)(dst3)
    dinv, gcur = _tc_prep(degp, x, W1)

    scat = _scatter_rows(n, e, h)
    b1r, gam1, bet1 = b1.reshape(1, h), g1.reshape(1, h), beta1.reshape(1, h)
    b2r, gam2, bet2 = b2.reshape(1, h), g2.reshape(1, h), beta2.reshape(1, h)
    b3r, gam3, bet3 = b3.reshape(1, h), g3.reshape(1, h), beta3.reshape(1, h)

    sp = scat(gcur, src3, dst3)
    gcur = _tc_mid(sp, gcur, dinv, b1r, gam1, bet1, W2)
    sp = scat(gcur, src3, dst3)
    gcur = _tc_mid(sp, gcur, dinv, b2r, gam2, bet2, W3)
    sp = scat(gcur, src3, dst3)

    batch2 = batch.reshape(n, 1)
    return _tc_final(sp, gcur, dinv, b3r, gam3, bet3, batch2,
                     Wl1, bl1.reshape(1, -1), Wl2, bl2.reshape(1, -1), G)


# speedup vs baseline: 39.0060x; 1.2401x over previous
import functools

import jax
import jax.numpy as jnp
from jax import lax
from jax.experimental import pallas as pl
from jax.experimental.pallas import tpu as pltpu
from jax.experimental.pallas import tpu_sc as plsc

_NTILE = 32
_K = 80
_ZR = 128


def _pad_rows(n):
    return -(-n // (16 * _ZR)) * (16 * _ZR)


def _scatter_rows(n, e, w):
    ntile = _NTILE
    ept = e // ntile
    nch = ept // _K
    npad = _pad_rows(n)
    rpt = npad // 16
    nz = rpt // _ZR
    assert nch % 2 == 1, "edge loop expects an odd chunk count (paired + tail)"
    mesh = plsc.VectorSubcoreMesh(core_axis_name="c", subcore_axis_name="s")

    @functools.partial(
        pl.kernel,
        mesh=mesh,
        out_type=jax.ShapeDtypeStruct((2, npad, w), jnp.float32),
        scratch_types=[
            pltpu.VMEM((nch, _K), jnp.int32),
            pltpu.VMEM((nch, _K), jnp.int32),
            pltpu.VMEM((4, _K, w), jnp.float32),
            pltpu.VMEM((_ZR, w), jnp.float32),
            pltpu.VMEM_SHARED((npad, w), jnp.float32),
            [pltpu.SemaphoreType.DMA] * 4,
            [pltpu.SemaphoreType.DMA] * 4,
            pltpu.SemaphoreType.DMA,
        ],
        compiler_params=pltpu.CompilerParams(use_tc_tiling_on_sc=False),
    )
    def k(g_hbm, src_hbm, dst_hbm, out_hbm, src_v, dst_v, rows_v, zer_v, acc_sh,
          gs, ss, isem):
        c = lax.axis_index("c")
        s = lax.axis_index("s")
        wid = c * 16 + s

        icp0 = pltpu.async_copy(src_hbm.at[wid], src_v, isem)
        icp1 = pltpu.async_copy(dst_hbm.at[wid], dst_v, isem)

        def zero_row(i, carry):
            for j in range(w // 16):
                zer_v[i, pl.ds(j * 16, 16)] = jnp.zeros((16,), jnp.float32)
            return carry
        lax.fori_loop(0, _ZR, zero_row, 0)

        base = s * rpt
        for r in range(nz):
            pltpu.sync_copy(zer_v, acc_sh.at[pl.ds(base + r * _ZR, _ZR)])
        icp0.wait()
        icp1.wait()
        plsc.subcore_barrier()

        def drain_g(b):
            pltpu.make_async_copy(g_hbm.at[pl.ds(0, _K)], rows_v.at[b], gs[b]).wait()

        def drain_s(b):
            pltpu.make_async_copy(g_hbm.at[pl.ds(0, _K)], rows_v.at[b], ss[b]).wait()

        pltpu.async_copy(g_hbm.at[src_v.at[0]], rows_v.at[0], gs[0])
        pltpu.async_copy(g_hbm.at[src_v.at[1]], rows_v.at[1], gs[1])

        nround = (nch - 1) // 4

        def round_body(p, carry):
            i0 = 4 * p
            for b in range(4):
                i = i0 + b
                bp = (b + 2) % 4
                if b < 2:
                    @pl.when(p >= 1)
                    def _():
                        drain_s(bp)
                    pltpu.async_copy(g_hbm.at[src_v.at[i + 2]], rows_v.at[bp], gs[bp])
                elif b == 2:
                    drain_s(bp)
                    pltpu.async_copy(g_hbm.at[src_v.at[i + 2]], rows_v.at[bp], gs[bp])
                else:
                    @pl.when(p < nround - 1)
                    def _():
                        drain_s(bp)
                        pltpu.async_copy(g_hbm.at[src_v.at[i + 2]], rows_v.at[bp], gs[bp])
                drain_g(b)
                pltpu.async_copy(rows_v.at[b], acc_sh.at[dst_v.at[i]], ss[b], add=True)
            return carry
        lax.fori_loop(0, nround, round_body, 0)

        drain_g(0)
        pltpu.async_copy(rows_v.at[0], acc_sh.at[dst_v.at[nch - 1]], ss[0], add=True)
        for b in range(4):
            drain_s(b)

        plsc.subcore_barrier()
        for r in range(nz):
            pltpu.sync_copy(acc_sh.at[pl.ds(base + r * _ZR, _ZR)],
                            out_hbm.at[c, pl.ds(base + r * _ZR, _ZR)])

    return k


def _deg_rows(n, e, w):
    ntile = _NTILE
    ept = e // ntile
    nch = ept // _K
    npad = _pad_rows(n)
    rpt = npad // 16
    nz = rpt // _ZR
    mesh = plsc.VectorSubcoreMesh(core_axis_name="c", subcore_axis_name="s")

    @functools.partial(
        pl.kernel,
        mesh=mesh,
        out_type=jax.ShapeDtypeStruct((2, npad, w), jnp.float32),
        scratch_types=[
            pltpu.VMEM((nch, _K), jnp.int32),
            pltpu.VMEM((_K, w), jnp.float32),
            pltpu.VMEM((_ZR, w), jnp.float32),
            pltpu.VMEM_SHARED((npad, w), jnp.float32),
            pltpu.SemaphoreType.DMA,
            [pltpu.SemaphoreType.DMA] * 2,
        ],
        compiler_params=pltpu.CompilerParams(use_tc_tiling_on_sc=False),
    )
    def k(dst_hbm, out_hbm, dst_v, ones_v, zer_v, acc_sh, isem, ss):
        c = lax.axis_index("c")
        s = lax.axis_index("s")
        wid = c * 16 + s
        icp = pltpu.async_copy(dst_hbm.at[wid], dst_v, isem)

        def zero_row(i, carry):
            for j in range(w // 16):
                zer_v[i, pl.ds(j * 16, 16)] = jnp.zeros((16,), jnp.float32)
            return carry
        lax.fori_loop(0, _ZR, zero_row, 0)

        def one_row(i, carry):
            for j in range(w // 16):
                ones_v[i, pl.ds(j * 16, 16)] = jnp.ones((16,), jnp.float32)
            return carry
        lax.fori_loop(0, _K, one_row, 0)

        base = s * rpt
        for r in range(nz):
            pltpu.sync_copy(zer_v, acc_sh.at[pl.ds(base + r * _ZR, _ZR)])
        icp.wait()
        plsc.subcore_barrier()

        def drain_s(b):
            pltpu.make_async_copy(out_hbm.at[0, pl.ds(0, _K)], ones_v, ss[b]).wait()

        def pair(p, carry):
            for b in range(2):
                @pl.when(p >= 1)
                def _():
                    drain_s(b)
                pltpu.async_copy(ones_v, acc_sh.at[dst_v.at[2 * p + b]], ss[b], add=True)
            return carry
        lax.fori_loop(0, nch // 2, pair, 0)
        drain_s(0)
        pltpu.async_copy(ones_v, acc_sh.at[dst_v.at[nch - 1]], ss[0], add=True)
        for b in range(2):
            drain_s(b)

        plsc.subcore_barrier()
        for r in range(nz):
            pltpu.sync_copy(acc_sh.at[pl.ds(base + r * _ZR, _ZR)],
                            out_hbm.at[c, pl.ds(base + r * _ZR, _ZR)])

    return k


def _tc_prep(degp, x, W1):
    n = x.shape[0]
    h = W1.shape[1]

    def body(degp_ref, x_ref, w1_ref, dinv_ref, g1_ref):
        deg = degp_ref[0, 0:n, 0:1] + degp_ref[1, 0:n, 0:1] + 1.0
        dinv = lax.rsqrt(deg)
        dinv_ref[...] = dinv
        g1_ref[...] = dinv * jnp.dot(x_ref[...], w1_ref[...],
                                     preferred_element_type=jnp.float32)

    return pl.pallas_call(
        body,
        out_shape=(jax.ShapeDtypeStruct((n, 1), jnp.float32),
                   jax.ShapeDtypeStruct((n, h), jnp.float32)),
    )(degp, x, W1)


def _tc_mid(sp, g, dinv, b, gam, bet, W):
    n, h = g.shape
    hn = W.shape[1]

    def body(sp_ref, g_ref, dinv_ref, b_ref, gam_ref, bet_ref, w_ref, gout_ref):
        dinv = dinv_ref[...]
        t = dinv * (sp_ref[0, 0:n, :] + sp_ref[1, 0:n, :] + g_ref[...]) + b_ref[...]
        mu = jnp.mean(t, axis=0, keepdims=True)
        xc = t - mu
        var = jnp.mean(xc * xc, axis=0, keepdims=True)
        hh = jnp.maximum(gam_ref[...] * xc * lax.rsqrt(var + 1e-5) + bet_ref[...], 0.0)
        gout_ref[...] = dinv * jnp.dot(hh, w_ref[...],
                                       preferred_element_type=jnp.float32)

    return pl.pallas_call(
        body,
        out_shape=jax.ShapeDtypeStruct((n, hn), jnp.float32),
    )(sp, g, dinv, b, gam, bet, W)


def _tc_final(sp, g, dinv, b, gam, bet, batch2, Wl1, bl1, Wl2, bl2, G):
    n, h = g.shape
    out = Wl2.shape[1]

    def body(sp_ref, g_ref, dinv_ref, b_ref, gam_ref, bet_ref, batch_ref,
             wl1_ref, bl1_ref, wl2_ref, bl2_ref, out_ref):
        dinv = dinv_ref[...]
        t = dinv * (sp_ref[0, 0:n, :] + sp_ref[1, 0:n, :] + g_ref[...]) + b_ref[...]
        mu = jnp.mean(t, axis=0, keepdims=True)
        xc = t - mu
        var = jnp.mean(xc * xc, axis=0, keepdims=True)
        hh = jnp.maximum(gam_ref[...] * xc * lax.rsqrt(var + 1e-5) + bet_ref[...], 0.0)

        gids = lax.broadcasted_iota(jnp.int32, (1, G), 1)
        m = (batch_ref[...] == gids).astype(jnp.float32)
        psum = lax.dot_general(m, hh, (((0,), (0,)), ((), ())),
                               preferred_element_type=jnp.float32)
        cnt = lax.dot_general(m, jnp.ones((n, 1), jnp.float32),
                              (((0,), (0,)), ((), ())),
                              preferred_element_type=jnp.float32)
        pooled = psum / jnp.maximum(cnt, 1.0)
        z = jnp.maximum(jnp.dot(pooled, wl1_ref[...],
                                preferred_element_type=jnp.float32) + bl1_ref[...], 0.0)
        out_ref[...] = jnp.dot(z, wl2_ref[...],
                               preferred_element_type=jnp.float32) + bl2_ref[...]

    return pl.pallas_call(
        body,
        out_shape=jax.ShapeDtypeStruct((G, out), jnp.float32),
    )(sp, g, dinv, b, gam, bet, batch2, Wl1, bl1, Wl2, bl2)


def kernel(x, edge_index, batch, W1, b1, g1, beta1, W2, b2, g2, beta2,
           W3, b3, g3, beta3, Wl1, bl1, Wl2, bl2):
    n, din = x.shape
    e = edge_index.shape[1]
    h = W1.shape[1]
    G = 64

    ept = e // _NTILE
    nch = ept // _K
    src3 = edge_index[0].reshape(_NTILE, nch, _K)
    dst3 = edge_index[1].reshape(_NTILE, nch, _K)

    degp = _deg_rows(n, e, 16)(dst3)
    dinv, gcur = _tc_prep(degp, x, W1)

    scat = _scatter_rows(n, e, h)
    b1r, gam1, bet1 = b1.reshape(1, h), g1.reshape(1, h), beta1.reshape(1, h)
    b2r, gam2, bet2 = b2.reshape(1, h), g2.reshape(1, h), beta2.reshape(1, h)
    b3r, gam3, bet3 = b3.reshape(1, h), g3.reshape(1, h), beta3.reshape(1, h)

    sp = scat(gcur, src3, dst3)
    gcur = _tc_mid(sp, gcur, dinv, b1r, gam1, bet1, W2)
    sp = scat(gcur, src3, dst3)
    gcur = _tc_mid(sp, gcur, dinv, b2r, gam2, bet2, W3)
    sp = scat(gcur, src3, dst3)

    batch2 = batch.reshape(n, 1)
    return _tc_final(sp, gcur, dinv, b3r, gam3, bet3, batch2,
                     Wl1, bl1.reshape(1, -1), Wl2, bl2.reshape(1, -1), G)
